# baseline (speedup 1.0000x reference)
import functools

import jax
import jax.numpy as jnp
from jax.experimental import pallas as pl

B, S, D = 1, 2048, 1024
H, KV, HD = 16, 4, 64
E, K, I = 8, 2, 512
EPS = 1e-6
THETA = 1000000.0
HALF = HD // 2
N_REP = H // KV

TS1 = 512
TSQ = 256
TS3 = 512
TSM = 512


def _rms(x, w, eps=EPS):
    nrm = jax.lax.rsqrt(jnp.mean(jnp.square(x), axis=-1, keepdims=True) + eps)
    return x * nrm * w


def _qkv_kernel(x_ref, nw_ref, wq_ref, wk_ref, wv_ref, q_ref, k_ref, v_ref):
    h = _rms(x_ref[...], nw_ref[...])
    q_ref[...] = jnp.dot(h, wq_ref[...], preferred_element_type=jnp.float32)
    k_ref[...] = jnp.dot(h, wk_ref[...], preferred_element_type=jnp.float32)
    v_ref[...] = jnp.dot(h, wv_ref[...], preferred_element_type=jnp.float32)


def _rope(x, cos, sin):
    x1 = x[:, :HALF]
    x2 = x[:, HALF:]
    return jnp.concatenate([x1 * cos - x2 * sin, x2 * cos + x1 * sin], axis=-1)


def _attn_kernel(q_ref, k_ref, v_ref, qc_ref, qs_ref, kc_ref, ks_ref,
                 qnw_ref, knw_ref, o_ref):
    i = pl.program_id(1)
    q = _rms(q_ref[0], qnw_ref[...])
    k = _rms(k_ref[0], knw_ref[...])
    q = _rope(q, qc_ref[...], qs_ref[...])
    k = _rope(k, kc_ref[...], ks_ref[...])
    scale = 1.0 / jnp.sqrt(jnp.float32(HD))
    s = jax.lax.dot_general(q, k, (((1,), (1,)), ((), ())),
                            preferred_element_type=jnp.float32) * scale
    row = i * TSQ + jax.lax.broadcasted_iota(jnp.int32, (TSQ, S), 0)
    col = jax.lax.broadcasted_iota(jnp.int32, (TSQ, S), 1)
    s = jnp.where(col <= row, s, jnp.float32(-1e30))
    m = jnp.max(s, axis=-1, keepdims=True)
    p = jnp.exp(s - m)
    p = p / jnp.sum(p, axis=-1, keepdims=True)
    o_ref[0] = jnp.dot(p, v_ref[0], preferred_element_type=jnp.float32)


def _post_kernel(ao_ref, wo_ref, x_ref, fw_ref, wg_ref, h2_ref, ht_ref, w_ref):
    h2 = x_ref[...] + jnp.dot(ao_ref[...], wo_ref[...],
                              preferred_element_type=jnp.float32)
    h2_ref[...] = h2
    ht = _rms(h2, fw_ref[...])
    ht_ref[...] = ht
    logits = jnp.dot(ht, wg_ref[...], preferred_element_type=jnp.float32)
    m = jnp.max(logits, axis=-1, keepdims=True)
    eg = jnp.exp(logits - m)
    gates = eg / jnp.sum(eg, axis=-1, keepdims=True)
    lane = jax.lax.broadcasted_iota(jnp.int32, gates.shape, 1)
    a1 = jnp.argmax(gates, axis=-1)[:, None]
    one1 = lane == a1
    v1 = jnp.max(gates, axis=-1, keepdims=True)
    g2 = jnp.where(one1, jnp.float32(-1.0), gates)
    a2 = jnp.argmax(g2, axis=-1)[:, None]
    one2 = lane == a2
    v2 = jnp.max(g2, axis=-1, keepdims=True)
    denom = jnp.maximum(v1 + v2, 1e-9)
    w_ref[...] = (jnp.where(one1, v1, 0.0) + jnp.where(one2, v2, 0.0)) / denom


def _moe_kernel(ht_ref, wgt_ref, wup_ref, wdn_ref, w_ref, h2_ref, o_ref):
    e = pl.program_id(1)
    ht = ht_ref[...]
    g = jnp.dot(ht, wgt_ref[0], preferred_element_type=jnp.float32)
    u = jnp.dot(ht, wup_ref[0], preferred_element_type=jnp.float32)
    inter = (g * jax.lax.logistic(g)) * u
    eo = jnp.dot(inter, wdn_ref[0], preferred_element_type=jnp.float32)
    lane = jax.lax.broadcasted_iota(jnp.int32, w_ref.shape, 1)
    wcol = jnp.sum(jnp.where(lane == e, w_ref[...], 0.0), axis=-1,
                   keepdims=True)
    contrib = wcol * eo

    @pl.when(e == 0)
    def _():
        o_ref[...] = h2_ref[...] + contrib

    @pl.when(e != 0)
    def _():
        o_ref[...] = o_ref[...] + contrib


def _rope_tables():
    freqs = 1.0 / (THETA ** (jnp.arange(0, HD, 2, dtype=jnp.float32) / HD))
    t = jnp.arange(S, dtype=jnp.float32)
    f = jnp.outer(t, freqs)
    return jnp.cos(f), jnp.sin(f)


@functools.partial(jax.jit, static_argnames=())
def kernel(hidden, attn_norm_w, q_norm_w, k_norm_w, ffn_norm_w, Wq, Wk, Wv,
           Wo, Wg, We_gate, We_up, We_down):
    x = hidden.reshape(S, D)
    cos, sin = _rope_tables()

    q, k, v = pl.pallas_call(
        _qkv_kernel,
        grid=(S // TS1,),
        in_specs=[
            pl.BlockSpec((TS1, D), lambda i: (i, 0)),
            pl.BlockSpec((1, D), lambda i: (0, 0)),
            pl.BlockSpec((D, H * HD), lambda i: (0, 0)),
            pl.BlockSpec((D, KV * HD), lambda i: (0, 0)),
            pl.BlockSpec((D, KV * HD), lambda i: (0, 0)),
        ],
        out_specs=[
            pl.BlockSpec((TS1, H * HD), lambda i: (i, 0)),
            pl.BlockSpec((TS1, KV * HD), lambda i: (i, 0)),
            pl.BlockSpec((TS1, KV * HD), lambda i: (i, 0)),
        ],
        out_shape=[
            jax.ShapeDtypeStruct((S, H * HD), jnp.float32),
            jax.ShapeDtypeStruct((S, KV * HD), jnp.float32),
            jax.ShapeDtypeStruct((S, KV * HD), jnp.float32),
        ],
    )(x, attn_norm_w.reshape(1, D), Wq, Wk, Wv)

    qh = q.reshape(S, H, HD).transpose(1, 0, 2)
    kh = k.reshape(S, KV, HD).transpose(1, 0, 2)
    vh = v.reshape(S, KV, HD).transpose(1, 0, 2)

    attn_out_h = pl.pallas_call(
        _attn_kernel,
        grid=(H, S // TSQ),
        in_specs=[
            pl.BlockSpec((1, TSQ, HD), lambda h, i: (h, i, 0)),
            pl.BlockSpec((1, S, HD), lambda h, i: (h // N_REP, 0, 0)),
            pl.BlockSpec((1, S, HD), lambda h, i: (h // N_REP, 0, 0)),
            pl.BlockSpec((TSQ, HALF), lambda h, i: (i, 0)),
            pl.BlockSpec((TSQ, HALF), lambda h, i: (i, 0)),
            pl.BlockSpec((S, HALF), lambda h, i: (0, 0)),
            pl.BlockSpec((S, HALF), lambda h, i: (0, 0)),
            pl.BlockSpec((1, HD), lambda h, i: (0, 0)),
            pl.BlockSpec((1, HD), lambda h, i: (0, 0)),
        ],
        out_specs=pl.BlockSpec((1, TSQ, HD), lambda h, i: (h, i, 0)),
        out_shape=jax.ShapeDtypeStruct((H, S, HD), jnp.float32),
    )(qh, kh, vh, cos, sin, cos, sin,
      q_norm_w.reshape(1, HD), k_norm_w.reshape(1, HD))
    attn_out = attn_out_h.transpose(1, 0, 2).reshape(S, H * HD)

    h2, ht, w = pl.pallas_call(
        _post_kernel,
        grid=(S // TS3,),
        in_specs=[
            pl.BlockSpec((TS3, H * HD), lambda i: (i, 0)),
            pl.BlockSpec((H * HD, D), lambda i: (0, 0)),
            pl.BlockSpec((TS3, D), lambda i: (i, 0)),
            pl.BlockSpec((1, D), lambda i: (0, 0)),
            pl.BlockSpec((D, E), lambda i: (0, 0)),
        ],
        out_specs=[
            pl.BlockSpec((TS3, D), lambda i: (i, 0)),
            pl.BlockSpec((TS3, D), lambda i: (i, 0)),
            pl.BlockSpec((TS3, E), lambda i: (i, 0)),
        ],
        out_shape=[
            jax.ShapeDtypeStruct((S, D), jnp.float32),
            jax.ShapeDtypeStruct((S, D), jnp.float32),
            jax.ShapeDtypeStruct((S, E), jnp.float32),
        ],
    )(attn_out, Wo, x, ffn_norm_w.reshape(1, D), Wg)

    out = pl.pallas_call(
        _moe_kernel,
        grid=(S // TSM, E),
        in_specs=[
            pl.BlockSpec((TSM, D), lambda t, e: (t, 0)),
            pl.BlockSpec((1, D, I), lambda t, e: (e, 0, 0)),
            pl.BlockSpec((1, D, I), lambda t, e: (e, 0, 0)),
            pl.BlockSpec((1, I, D), lambda t, e: (e, 0, 0)),
            pl.BlockSpec((TSM, E), lambda t, e: (t, 0)),
            pl.BlockSpec((TSM, D), lambda t, e: (t, 0)),
        ],
        out_specs=pl.BlockSpec((TSM, D), lambda t, e: (t, 0)),
        out_shape=jax.ShapeDtypeStruct((S, D), jnp.float32),
    )(ht, We_gate, We_up, We_down, w, h2)

    return out.reshape(B, S, D)


# speedup vs baseline: 1.2701x; 1.2701x over previous
import functools

import jax
import jax.numpy as jnp
from jax.experimental import pallas as pl

B, S, D = 1, 2048, 1024
H, KV, HD = 16, 4, 64
E, K, I = 8, 2, 512
EPS = 1e-6
THETA = 1000000.0
HALF = HD // 2
N_REP = H // KV

TS1 = 512
TSQ = 256
TS3 = 512
TSM = 512


def _rms(x, w, eps=EPS):
    nrm = jax.lax.rsqrt(jnp.mean(jnp.square(x), axis=-1, keepdims=True) + eps)
    return x * nrm * w


def _qkv_kernel(x_ref, nw_ref, wq_ref, wk_ref, wv_ref, q_ref, k_ref, v_ref):
    h = _rms(x_ref[...], nw_ref[...]).astype(jnp.bfloat16)
    q_ref[...] = jnp.dot(h, wq_ref[...].astype(jnp.bfloat16),
                         preferred_element_type=jnp.float32)
    k_ref[...] = jnp.dot(h, wk_ref[...].astype(jnp.bfloat16),
                         preferred_element_type=jnp.float32)
    v_ref[...] = jnp.dot(h, wv_ref[...].astype(jnp.bfloat16),
                         preferred_element_type=jnp.float32)


def _rope(x, cos, sin):
    x1 = x[:, :HALF]
    x2 = x[:, HALF:]
    return jnp.concatenate([x1 * cos - x2 * sin, x2 * cos + x1 * sin], axis=-1)


def _attn_kernel(q_ref, k_ref, v_ref, qc_ref, qs_ref, kc_ref, ks_ref,
                 qnw_ref, knw_ref, o_ref):
    i = pl.program_id(1)
    q = _rms(q_ref[0], qnw_ref[...])
    k = _rms(k_ref[0], knw_ref[...])
    q = _rope(q, qc_ref[...], qs_ref[...])
    k = _rope(k, kc_ref[...], ks_ref[...])
    scale = 1.0 / jnp.sqrt(jnp.float32(HD))
    s = jax.lax.dot_general(q.astype(jnp.bfloat16), k.astype(jnp.bfloat16),
                            (((1,), (1,)), ((), ())),
                            preferred_element_type=jnp.float32) * scale
    row = i * TSQ + jax.lax.broadcasted_iota(jnp.int32, (TSQ, S), 0)
    col = jax.lax.broadcasted_iota(jnp.int32, (TSQ, S), 1)
    s = jnp.where(col <= row, s, jnp.float32(-1e30))
    m = jnp.max(s, axis=-1, keepdims=True)
    p = jnp.exp(s - m)
    p = p / jnp.sum(p, axis=-1, keepdims=True)
    o_ref[0] = jnp.dot(p.astype(jnp.bfloat16), v_ref[0].astype(jnp.bfloat16),
                       preferred_element_type=jnp.float32)


def _post_kernel(ao_ref, wo_ref, x_ref, fw_ref, wg_ref, h2_ref, ht_ref, w_ref):
    h2 = x_ref[...] + jnp.dot(ao_ref[...].astype(jnp.bfloat16),
                              wo_ref[...].astype(jnp.bfloat16),
                              preferred_element_type=jnp.float32)
    h2_ref[...] = h2
    ht = _rms(h2, fw_ref[...])
    ht_ref[...] = ht
    logits = jnp.dot(ht.astype(jnp.bfloat16), wg_ref[...].astype(jnp.bfloat16),
                     preferred_element_type=jnp.float32)
    m = jnp.max(logits, axis=-1, keepdims=True)
    eg = jnp.exp(logits - m)
    gates = eg / jnp.sum(eg, axis=-1, keepdims=True)
    lane = jax.lax.broadcasted_iota(jnp.int32, gates.shape, 1)
    a1 = jnp.argmax(gates, axis=-1)[:, None]
    one1 = lane == a1
    v1 = jnp.max(gates, axis=-1, keepdims=True)
    g2 = jnp.where(one1, jnp.float32(-1.0), gates)
    a2 = jnp.argmax(g2, axis=-1)[:, None]
    one2 = lane == a2
    v2 = jnp.max(g2, axis=-1, keepdims=True)
    denom = jnp.maximum(v1 + v2, 1e-9)
    w_ref[...] = (jnp.where(one1, v1, 0.0) + jnp.where(one2, v2, 0.0)) / denom


def _moe_kernel(ht_ref, wgt_ref, wup_ref, wdn_ref, w_ref, h2_ref, o_ref):
    e = pl.program_id(1)
    ht = ht_ref[...].astype(jnp.bfloat16)
    g = jnp.dot(ht, wgt_ref[0].astype(jnp.bfloat16),
                preferred_element_type=jnp.float32)
    u = jnp.dot(ht, wup_ref[0].astype(jnp.bfloat16),
                preferred_element_type=jnp.float32)
    inter = (g * jax.lax.logistic(g)) * u
    eo = jnp.dot(inter.astype(jnp.bfloat16), wdn_ref[0].astype(jnp.bfloat16),
                 preferred_element_type=jnp.float32)
    lane = jax.lax.broadcasted_iota(jnp.int32, w_ref.shape, 1)
    wcol = jnp.sum(jnp.where(lane == e, w_ref[...], 0.0), axis=-1,
                   keepdims=True)
    contrib = wcol * eo

    @pl.when(e == 0)
    def _():
        o_ref[...] = h2_ref[...] + contrib

    @pl.when(e != 0)
    def _():
        o_ref[...] = o_ref[...] + contrib


def _rope_tables():
    freqs = 1.0 / (THETA ** (jnp.arange(0, HD, 2, dtype=jnp.float32) / HD))
    t = jnp.arange(S, dtype=jnp.float32)
    f = jnp.outer(t, freqs)
    return jnp.cos(f), jnp.sin(f)


@functools.partial(jax.jit, static_argnames=())
def kernel(hidden, attn_norm_w, q_norm_w, k_norm_w, ffn_norm_w, Wq, Wk, Wv,
           Wo, Wg, We_gate, We_up, We_down):
    x = hidden.reshape(S, D)
    cos, sin = _rope_tables()

    q, k, v = pl.pallas_call(
        _qkv_kernel,
        grid=(S // TS1,),
        in_specs=[
            pl.BlockSpec((TS1, D), lambda i: (i, 0)),
            pl.BlockSpec((1, D), lambda i: (0, 0)),
            pl.BlockSpec((D, H * HD), lambda i: (0, 0)),
            pl.BlockSpec((D, KV * HD), lambda i: (0, 0)),
            pl.BlockSpec((D, KV * HD), lambda i: (0, 0)),
        ],
        out_specs=[
            pl.BlockSpec((TS1, H * HD), lambda i: (i, 0)),
            pl.BlockSpec((TS1, KV * HD), lambda i: (i, 0)),
            pl.BlockSpec((TS1, KV * HD), lambda i: (i, 0)),
        ],
        out_shape=[
            jax.ShapeDtypeStruct((S, H * HD), jnp.float32),
            jax.ShapeDtypeStruct((S, KV * HD), jnp.float32),
            jax.ShapeDtypeStruct((S, KV * HD), jnp.float32),
        ],
    )(x, attn_norm_w.reshape(1, D), Wq, Wk, Wv)

    qh = q.reshape(S, H, HD).transpose(1, 0, 2)
    kh = k.reshape(S, KV, HD).transpose(1, 0, 2)
    vh = v.reshape(S, KV, HD).transpose(1, 0, 2)

    attn_out_h = pl.pallas_call(
        _attn_kernel,
        grid=(H, S // TSQ),
        in_specs=[
            pl.BlockSpec((1, TSQ, HD), lambda h, i: (h, i, 0)),
            pl.BlockSpec((1, S, HD), lambda h, i: (h // N_REP, 0, 0)),
            pl.BlockSpec((1, S, HD), lambda h, i: (h // N_REP, 0, 0)),
            pl.BlockSpec((TSQ, HALF), lambda h, i: (i, 0)),
            pl.BlockSpec((TSQ, HALF), lambda h, i: (i, 0)),
            pl.BlockSpec((S, HALF), lambda h, i: (0, 0)),
            pl.BlockSpec((S, HALF), lambda h, i: (0, 0)),
            pl.BlockSpec((1, HD), lambda h, i: (0, 0)),
            pl.BlockSpec((1, HD), lambda h, i: (0, 0)),
        ],
        out_specs=pl.BlockSpec((1, TSQ, HD), lambda h, i: (h, i, 0)),
        out_shape=jax.ShapeDtypeStruct((H, S, HD), jnp.float32),
    )(qh, kh, vh, cos, sin, cos, sin,
      q_norm_w.reshape(1, HD), k_norm_w.reshape(1, HD))
    attn_out = attn_out_h.transpose(1, 0, 2).reshape(S, H * HD)

    h2, ht, w = pl.pallas_call(
        _post_kernel,
        grid=(S // TS3,),
        in_specs=[
            pl.BlockSpec((TS3, H * HD), lambda i: (i, 0)),
            pl.BlockSpec((H * HD, D), lambda i: (0, 0)),
            pl.BlockSpec((TS3, D), lambda i: (i, 0)),
            pl.BlockSpec((1, D), lambda i: (0, 0)),
            pl.BlockSpec((D, E), lambda i: (0, 0)),
        ],
        out_specs=[
            pl.BlockSpec((TS3, D), lambda i: (i, 0)),
            pl.BlockSpec((TS3, D), lambda i: (i, 0)),
            pl.BlockSpec((TS3, E), lambda i: (i, 0)),
        ],
        out_shape=[
            jax.ShapeDtypeStruct((S, D), jnp.float32),
            jax.ShapeDtypeStruct((S, D), jnp.float32),
            jax.ShapeDtypeStruct((S, E), jnp.float32),
        ],
    )(attn_out, Wo, x, ffn_norm_w.reshape(1, D), Wg)

    out = pl.pallas_call(
        _moe_kernel,
        grid=(S // TSM, E),
        in_specs=[
            pl.BlockSpec((TSM, D), lambda t, e: (t, 0)),
            pl.BlockSpec((1, D, I), lambda t, e: (e, 0, 0)),
            pl.BlockSpec((1, D, I), lambda t, e: (e, 0, 0)),
            pl.BlockSpec((1, I, D), lambda t, e: (e, 0, 0)),
            pl.BlockSpec((TSM, E), lambda t, e: (t, 0)),
            pl.BlockSpec((TSM, D), lambda t, e: (t, 0)),
        ],
        out_specs=pl.BlockSpec((TSM, D), lambda t, e: (t, 0)),
        out_shape=jax.ShapeDtypeStruct((S, D), jnp.float32),
    )(ht, We_gate, We_up, We_down, w, h2)

    return out.reshape(B, S, D)


# speedup vs baseline: 1.6636x; 1.3099x over previous
import functools

import jax
import jax.numpy as jnp
from jax.experimental import pallas as pl

B, S, D = 1, 2048, 1024
H, KV, HD = 16, 4, 64
E, K, I = 8, 2, 512
EPS = 1e-6
THETA = 1000000.0
HALF = HD // 2
N_REP = H // KV
SCALE = 1.0 / 8.0

TS1 = 512
TSQ = 256
TS3 = 512
TSM = 512


def _rms(x, w, eps=EPS):
    nrm = jax.lax.rsqrt(jnp.mean(jnp.square(x), axis=-1, keepdims=True) + eps)
    return x * nrm * w


def _rope(x, cos, sin):
    x1 = x[:, :HALF]
    x2 = x[:, HALF:]
    return jnp.concatenate([x1 * cos - x2 * sin, x2 * cos + x1 * sin], axis=-1)


def _qkv_kernel(x_ref, nw_ref, wq_ref, wk_ref, wv_ref, c_ref, s_ref,
                qnw_ref, knw_ref, q_ref, k_ref, v_ref):
    h = _rms(x_ref[...], nw_ref[...]).astype(jnp.bfloat16)
    q = jnp.dot(h, wq_ref[...].astype(jnp.bfloat16),
                preferred_element_type=jnp.float32)
    k = jnp.dot(h, wk_ref[...].astype(jnp.bfloat16),
                preferred_element_type=jnp.float32)
    v = jnp.dot(h, wv_ref[...].astype(jnp.bfloat16),
                preferred_element_type=jnp.float32)
    cos = c_ref[...]
    sin = s_ref[...]
    for hh in range(H):
        qh = _rms(q[:, hh * HD:(hh + 1) * HD], qnw_ref[...])
        qh = _rope(qh, cos, sin) * SCALE
        q_ref[hh] = qh.astype(jnp.bfloat16)
    for g in range(KV):
        kg = _rms(k[:, g * HD:(g + 1) * HD], knw_ref[...])
        kg = _rope(kg, cos, sin)
        k_ref[g] = kg.astype(jnp.bfloat16)
        v_ref[g] = v[:, g * HD:(g + 1) * HD].astype(jnp.bfloat16)


def _attn_kernel(q_ref, k_ref, v_ref, o_ref):
    i = pl.program_id(1)
    q = q_ref[0]
    acc0 = jnp.zeros((TSQ, HD), jnp.float32)
    esum0 = jnp.zeros((TSQ, TSQ), jnp.float32)

    def body(j, carry):
        acc, esum = carry
        base = pl.multiple_of(j * TSQ, TSQ)
        kj = k_ref[0, pl.ds(base, TSQ), :]
        vj = v_ref[0, pl.ds(base, TSQ), :]
        s = jax.lax.dot_general(q, kj, (((1,), (1,)), ((), ())),
                                preferred_element_type=jnp.float32)
        e = jnp.exp(s - 8.0)
        acc = acc + jnp.dot(e.astype(jnp.bfloat16), vj,
                            preferred_element_type=jnp.float32)
        return acc, esum + e

    acc, esum = jax.lax.fori_loop(0, i, body, (acc0, esum0))

    base = pl.multiple_of(i * TSQ, TSQ)
    kd = k_ref[0, pl.ds(base, TSQ), :]
    vd = v_ref[0, pl.ds(base, TSQ), :]
    s = jax.lax.dot_general(q, kd, (((1,), (1,)), ((), ())),
                            preferred_element_type=jnp.float32)
    row = jax.lax.broadcasted_iota(jnp.int32, (TSQ, TSQ), 0)
    col = jax.lax.broadcasted_iota(jnp.int32, (TSQ, TSQ), 1)
    e = jnp.where(col <= row, jnp.exp(s - 8.0), 0.0)
    acc = acc + jnp.dot(e.astype(jnp.bfloat16), vd,
                        preferred_element_type=jnp.float32)
    esum = esum + e
    denom = jnp.sum(esum, axis=-1, keepdims=True)
    o_ref[0] = acc * (1.0 / denom)


def _post_kernel(ao_ref, wo_ref, x_ref, fw_ref, wg_ref, h2_ref, ht_ref, w_ref):
    h2 = x_ref[...] + jnp.dot(ao_ref[...].astype(jnp.bfloat16),
                              wo_ref[...].astype(jnp.bfloat16),
                              preferred_element_type=jnp.float32)
    h2_ref[...] = h2
    ht = _rms(h2, fw_ref[...])
    ht_ref[...] = ht
    logits = jnp.dot(ht.astype(jnp.bfloat16), wg_ref[...].astype(jnp.bfloat16),
                     preferred_element_type=jnp.float32)
    m = jnp.max(logits, axis=-1, keepdims=True)
    eg = jnp.exp(logits - m)
    gates = eg / jnp.sum(eg, axis=-1, keepdims=True)
    lane = jax.lax.broadcasted_iota(jnp.int32, gates.shape, 1)
    a1 = jnp.argmax(gates, axis=-1)[:, None]
    one1 = lane == a1
    v1 = jnp.max(gates, axis=-1, keepdims=True)
    g2 = jnp.where(one1, jnp.float32(-1.0), gates)
    a2 = jnp.argmax(g2, axis=-1)[:, None]
    one2 = lane == a2
    v2 = jnp.max(g2, axis=-1, keepdims=True)
    denom = jnp.maximum(v1 + v2, 1e-9)
    w_ref[...] = (jnp.where(one1, v1, 0.0) + jnp.where(one2, v2, 0.0)) / denom


def _moe_kernel(ht_ref, wgt_ref, wup_ref, wdn_ref, w_ref, h2_ref, o_ref):
    e = pl.program_id(1)
    ht = ht_ref[...].astype(jnp.bfloat16)
    g = jnp.dot(ht, wgt_ref[0].astype(jnp.bfloat16),
                preferred_element_type=jnp.float32)
    u = jnp.dot(ht, wup_ref[0].astype(jnp.bfloat16),
                preferred_element_type=jnp.float32)
    inter = (g * jax.lax.logistic(g)) * u
    eo = jnp.dot(inter.astype(jnp.bfloat16), wdn_ref[0].astype(jnp.bfloat16),
                 preferred_element_type=jnp.float32)
    lane = jax.lax.broadcasted_iota(jnp.int32, w_ref.shape, 1)
    wcol = jnp.sum(jnp.where(lane == e, w_ref[...], 0.0), axis=-1,
                   keepdims=True)
    contrib = wcol * eo

    @pl.when(e == 0)
    def _():
        o_ref[...] = h2_ref[...] + contrib

    @pl.when(e != 0)
    def _():
        o_ref[...] = o_ref[...] + contrib


def _rope_tables():
    freqs = 1.0 / (THETA ** (jnp.arange(0, HD, 2, dtype=jnp.float32) / HD))
    t = jnp.arange(S, dtype=jnp.float32)
    f = jnp.outer(t, freqs)
    return jnp.cos(f), jnp.sin(f)


@functools.partial(jax.jit, static_argnames=())
def kernel(hidden, attn_norm_w, q_norm_w, k_norm_w, ffn_norm_w, Wq, Wk, Wv,
           Wo, Wg, We_gate, We_up, We_down):
    x = hidden.reshape(S, D)
    cos, sin = _rope_tables()

    qh, kh, vh = pl.pallas_call(
        _qkv_kernel,
        grid=(S // TS1,),
        in_specs=[
            pl.BlockSpec((TS1, D), lambda i: (i, 0)),
            pl.BlockSpec((1, D), lambda i: (0, 0)),
            pl.BlockSpec((D, H * HD), lambda i: (0, 0)),
            pl.BlockSpec((D, KV * HD), lambda i: (0, 0)),
            pl.BlockSpec((D, KV * HD), lambda i: (0, 0)),
            pl.BlockSpec((TS1, HALF), lambda i: (i, 0)),
            pl.BlockSpec((TS1, HALF), lambda i: (i, 0)),
            pl.BlockSpec((1, HD), lambda i: (0, 0)),
            pl.BlockSpec((1, HD), lambda i: (0, 0)),
        ],
        out_specs=[
            pl.BlockSpec((H, TS1, HD), lambda i: (0, i, 0)),
            pl.BlockSpec((KV, TS1, HD), lambda i: (0, i, 0)),
            pl.BlockSpec((KV, TS1, HD), lambda i: (0, i, 0)),
        ],
        out_shape=[
            jax.ShapeDtypeStruct((H, S, HD), jnp.bfloat16),
            jax.ShapeDtypeStruct((KV, S, HD), jnp.bfloat16),
            jax.ShapeDtypeStruct((KV, S, HD), jnp.bfloat16),
        ],
    )(x, attn_norm_w.reshape(1, D), Wq, Wk, Wv, cos, sin,
      q_norm_w.reshape(1, HD), k_norm_w.reshape(1, HD))

    attn_out_h = pl.pallas_call(
        _attn_kernel,
        grid=(H, S // TSQ),
        in_specs=[
            pl.BlockSpec((1, TSQ, HD), lambda h, i: (h, i, 0)),
            pl.BlockSpec((1, S, HD), lambda h, i: (h // N_REP, 0, 0)),
            pl.BlockSpec((1, S, HD), lambda h, i: (h // N_REP, 0, 0)),
        ],
        out_specs=pl.BlockSpec((1, TSQ, HD), lambda h, i: (h, i, 0)),
        out_shape=jax.ShapeDtypeStruct((H, S, HD), jnp.float32),
    )(qh, kh, vh)
    attn_out = attn_out_h.transpose(1, 0, 2).reshape(S, H * HD)

    h2, ht, w = pl.pallas_call(
        _post_kernel,
        grid=(S // TS3,),
        in_specs=[
            pl.BlockSpec((TS3, H * HD), lambda i: (i, 0)),
            pl.BlockSpec((H * HD, D), lambda i: (0, 0)),
            pl.BlockSpec((TS3, D), lambda i: (i, 0)),
            pl.BlockSpec((1, D), lambda i: (0, 0)),
            pl.BlockSpec((D, E), lambda i: (0, 0)),
        ],
        out_specs=[
            pl.BlockSpec((TS3, D), lambda i: (i, 0)),
            pl.BlockSpec((TS3, D), lambda i: (i, 0)),
            pl.BlockSpec((TS3, E), lambda i: (i, 0)),
        ],
        out_shape=[
            jax.ShapeDtypeStruct((S, D), jnp.float32),
            jax.ShapeDtypeStruct((S, D), jnp.float32),
            jax.ShapeDtypeStruct((S, E), jnp.float32),
        ],
    )(attn_out, Wo, x, ffn_norm_w.reshape(1, D), Wg)

    out = pl.pallas_call(
        _moe_kernel,
        grid=(S // TSM, E),
        in_specs=[
            pl.BlockSpec((TSM, D), lambda t, e: (t, 0)),
            pl.BlockSpec((1, D, I), lambda t, e: (e, 0, 0)),
            pl.BlockSpec((1, D, I), lambda t, e: (e, 0, 0)),
            pl.BlockSpec((1, I, D), lambda t, e: (e, 0, 0)),
            pl.BlockSpec((TSM, E), lambda t, e: (t, 0)),
            pl.BlockSpec((TSM, D), lambda t, e: (t, 0)),
        ],
        out_specs=pl.BlockSpec((TSM, D), lambda t, e: (t, 0)),
        out_shape=jax.ShapeDtypeStruct((S, D), jnp.float32),
    )(ht, We_gate, We_up, We_down, w, h2)

    return out.reshape(B, S, D)
